# baseline (speedup 1.0000x reference)
import functools

import jax
import jax.numpy as jnp
from jax import lax
from jax.experimental import pallas as pl
from jax.experimental.pallas import tpu as pltpu
from jax.experimental.pallas import tpu_sc as plsc

N_ROWS = 100000
N_COLS = 512
LANES = 16

R_TC = 80000
R_SC = N_ROWS - R_TC
BLOCK_ROWS = 10000
NUM_BLOCKS = R_TC // BLOCK_ROWS

SC_COLS = 16
SC_CHUNK = 2000
N_ACC = 8

BINS_PER_TILE = 3136
N_BINS_PAD = 32 * BINS_PER_TILE
N_SUBCORES = 16
COLS_PER_SUBCORE = N_COLS // N_SUBCORES
ZERO_PER_TILE = N_BINS_PAD // N_SUBCORES
BINS_PER_CORE = N_BINS_PAD // 2
INV_TOTAL = 1.0 / N_COLS


def _maxpool_body(x_ref, max_out, idx_out):
    i = pl.program_id(0)
    x = x_ref[...]
    bmax = jnp.max(x, axis=0, keepdims=True)
    rows = lax.broadcasted_iota(jnp.int32, x.shape, 0) + i * BLOCK_ROWS
    bidx = jnp.min(
        jnp.where(x == bmax, rows, jnp.int32(N_ROWS)), axis=0, keepdims=True
    )

    @pl.when(i == 0)
    def _():
        max_out[...] = bmax
        idx_out[...] = bidx

    @pl.when(i > 0)
    def _():
        better = bmax > max_out[...]
        idx_out[...] = jnp.where(better, bidx, idx_out[...])
        max_out[...] = jnp.where(better, bmax, max_out[...])


_maxpool = pl.pallas_call(
    _maxpool_body,
    grid=(NUM_BLOCKS,),
    in_specs=[pl.BlockSpec((BLOCK_ROWS, N_COLS), lambda i: (i, 0))],
    out_specs=[
        pl.BlockSpec((1, N_COLS), lambda i: (0, 0)),
        pl.BlockSpec((1, N_COLS), lambda i: (0, 0)),
    ],
    out_shape=[
        jax.ShapeDtypeStruct((1, N_COLS), jnp.float32),
        jax.ShapeDtypeStruct((1, N_COLS), jnp.int32),
    ],
)


def _scmax_body(feat_hbm, max_out_hbm, idx_out_hbm, buf_v, max_v, idx_v):
    cid = lax.axis_index("c")
    sid = lax.axis_index("s")
    wid = sid * 2 + cid
    col0 = wid * SC_COLS

    def chunk_step(ci, carry):
        pltpu.sync_copy(
            feat_hbm.at[pl.ds(R_TC + ci * SC_CHUNK, SC_CHUNK), pl.ds(col0, SC_COLS)],
            buf_v,
        )
        row0 = R_TC + ci * SC_CHUNK
        lane_iota = lax.iota(jnp.int32, LANES)

        def row_step(t, acc):
            new = []
            for j in range(N_ACC):
                am, ai = acc[j]
                rowv = jnp.full((LANES,), t * N_ACC + j, jnp.int32)
                x = plsc.load_gather(buf_v, [rowv, lane_iota])
                rid = rowv + row0
                m = x > am
                new.append((jnp.where(m, x, am), jnp.where(m, rid, ai)))
            return tuple(new)

        return lax.fori_loop(0, SC_CHUNK // N_ACC, row_step, carry)

    init = tuple(
        (jnp.full((LANES,), -jnp.inf, jnp.float32), jnp.zeros((LANES,), jnp.int32))
        for _ in range(N_ACC)
    )
    accs = lax.fori_loop(0, R_SC // SC_CHUNK, chunk_step, init)
    fm, fi = accs[0]
    for j in range(1, N_ACC):
        bm, bi = accs[j]
        take = (bm > fm) | ((bm == fm) & (bi < fi))
        fm = jnp.where(take, bm, fm)
        fi = jnp.where(take, bi, fi)
    max_v[...] = fm
    idx_v[...] = fi
    pltpu.sync_copy(max_v, max_out_hbm.at[pl.ds(col0, SC_COLS)])
    pltpu.sync_copy(idx_v, idx_out_hbm.at[pl.ds(col0, SC_COLS)])


@functools.cache
def _scmax():
    return functools.partial(
        pl.kernel,
        mesh=plsc.VectorSubcoreMesh(core_axis_name="c", subcore_axis_name="s"),
        out_type=[
            jax.ShapeDtypeStruct((N_COLS,), jnp.float32),
            jax.ShapeDtypeStruct((N_COLS,), jnp.int32),
        ],
        scratch_types=[
            pltpu.VMEM((SC_CHUNK, SC_COLS), jnp.float32),
            pltpu.VMEM((SC_COLS,), jnp.float32),
            pltpu.VMEM((SC_COLS,), jnp.int32),
        ],
        compiler_params=pltpu.CompilerParams(
            use_tc_tiling_on_sc=False, needs_layout_passes=False
        ),
    )(_scmax_body)


def _hist_body(
    tcm_hbm, tci_hbm, scm_hbm, sci_hbm, out_hbm, pool_hbm,
    tcm_v, tci_v, scm_v, sci_v, cmb_max_v, cmb_idx_v, vals_v, zeros_v, hist_sh,
):
    cid = lax.axis_index("c")
    sid = lax.axis_index("s")
    zeros16 = jnp.zeros((LANES,), jnp.float32)
    for i in range(ZERO_PER_TILE // LANES):
        zeros_v[pl.ds(i * LANES, LANES)] = zeros16
    vals16 = jnp.full((LANES,), INV_TOTAL, jnp.float32)
    for i in range(COLS_PER_SUBCORE // LANES):
        vals_v[pl.ds(i * LANES, LANES)] = vals16
    col0 = sid * COLS_PER_SUBCORE
    pltpu.sync_copy(tcm_hbm.at[pl.ds(col0, COLS_PER_SUBCORE)], tcm_v)
    pltpu.sync_copy(tci_hbm.at[pl.ds(col0, COLS_PER_SUBCORE)], tci_v)
    pltpu.sync_copy(scm_hbm.at[pl.ds(col0, COLS_PER_SUBCORE)], scm_v)
    pltpu.sync_copy(sci_hbm.at[pl.ds(col0, COLS_PER_SUBCORE)], sci_v)
    for h in range(COLS_PER_SUBCORE // LANES):
        s = pl.ds(h * LANES, LANES)
        tm = tcm_v[s]
        ti = tci_v[s]
        sm = scm_v[s]
        si = sci_v[s]
        take = sm > tm
        cmb_max_v[s] = jnp.where(take, sm, tm)
        cmb_idx_v[s] = jnp.where(take, si, ti)
    pltpu.sync_copy(zeros_v, hist_sh.at[pl.ds(sid * ZERO_PER_TILE, ZERO_PER_TILE)])
    plsc.subcore_barrier()
    pltpu.sync_copy(vals_v, hist_sh.at[cmb_idx_v], add=True)
    plsc.subcore_barrier()
    goff = cid * BINS_PER_CORE + sid * BINS_PER_TILE
    out_v = zeros_v.at[pl.ds(0, BINS_PER_TILE)]
    pltpu.sync_copy(hist_sh.at[pl.ds(goff, BINS_PER_TILE)], out_v)
    pltpu.sync_copy(out_v, out_hbm.at[pl.ds(goff, BINS_PER_TILE)])

    @pl.when(cid == 0)
    def _():
        pltpu.sync_copy(cmb_max_v, pool_hbm.at[pl.ds(col0, COLS_PER_SUBCORE)])


@functools.cache
def _hist():
    return functools.partial(
        pl.kernel,
        mesh=plsc.VectorSubcoreMesh(core_axis_name="c", subcore_axis_name="s"),
        out_type=[
            jax.ShapeDtypeStruct((N_BINS_PAD,), jnp.float32),
            jax.ShapeDtypeStruct((N_COLS,), jnp.float32),
        ],
        scratch_types=[
            pltpu.VMEM((COLS_PER_SUBCORE,), jnp.float32),
            pltpu.VMEM((COLS_PER_SUBCORE,), jnp.int32),
            pltpu.VMEM((COLS_PER_SUBCORE,), jnp.float32),
            pltpu.VMEM((COLS_PER_SUBCORE,), jnp.int32),
            pltpu.VMEM((COLS_PER_SUBCORE,), jnp.float32),
            pltpu.VMEM((COLS_PER_SUBCORE,), jnp.int32),
            pltpu.VMEM((COLS_PER_SUBCORE,), jnp.float32),
            pltpu.VMEM((ZERO_PER_TILE,), jnp.float32),
            pltpu.VMEM_SHARED((N_BINS_PAD,), jnp.float32),
        ],
    )(_hist_body)


@jax.jit
def kernel(features):
    tc_max, tc_idx = _maxpool(features)
    sc_max, sc_idx = _scmax()(features)
    hist, pooled = _hist()(
        tc_max.reshape(N_COLS), tc_idx.reshape(N_COLS), sc_max, sc_idx
    )
    attention_weights = hist[:N_ROWS].reshape(1, N_ROWS)
    return (attention_weights, pooled.reshape(1, N_COLS))


# speedup vs baseline: 2.6561x; 2.6561x over previous
import functools

import jax
import jax.numpy as jnp
from jax import lax
from jax.experimental import pallas as pl
from jax.experimental.pallas import tpu as pltpu
from jax.experimental.pallas import tpu_sc as plsc

N_ROWS = 100000
N_COLS = 512
BLOCK_ROWS = 10000
NUM_BLOCKS = N_ROWS // BLOCK_ROWS

BINS_PER_TILE = 3136
N_BINS_PAD = 32 * BINS_PER_TILE
LANES = 16
INV_TOTAL = 1.0 / N_COLS

N_SUBCORES = 16
IDX_PER_TILE = N_COLS // N_SUBCORES
ZERO_PER_TILE = N_BINS_PAD // N_SUBCORES
BINS_PER_CORE = N_BINS_PAD // 2


def _maxpool_body(x_ref, max_out, idx_out):
    i = pl.program_id(0)
    x = x_ref[...]
    bmax = jnp.max(x, axis=0, keepdims=True)
    rows = lax.broadcasted_iota(jnp.int32, x.shape, 0) + i * BLOCK_ROWS
    bidx = jnp.min(
        jnp.where(x == bmax, rows, jnp.int32(N_ROWS)), axis=0, keepdims=True
    )

    @pl.when(i == 0)
    def _():
        max_out[...] = bmax
        idx_out[...] = bidx

    @pl.when(i > 0)
    def _():
        better = bmax > max_out[...]
        idx_out[...] = jnp.where(better, bidx, idx_out[...])
        max_out[...] = jnp.where(better, bmax, max_out[...])


_maxpool = pl.pallas_call(
    _maxpool_body,
    grid=(NUM_BLOCKS,),
    in_specs=[pl.BlockSpec((BLOCK_ROWS, N_COLS), lambda i: (i, 0))],
    out_specs=[
        pl.BlockSpec((1, N_COLS), lambda i: (0, 0)),
        pl.BlockSpec((1, N_COLS), lambda i: (0, 0)),
    ],
    out_shape=[
        jax.ShapeDtypeStruct((1, N_COLS), jnp.float32),
        jax.ShapeDtypeStruct((1, N_COLS), jnp.int32),
    ],
)


def _hist_body(idx_hbm, out_hbm, idx_row_v, vals_v, zeros_v, hist_sh, sem):
    cid = lax.axis_index("c")
    sid = lax.axis_index("s")
    idx_cp = pltpu.async_copy(idx_hbm.at[sid], idx_row_v, sem)
    zeros16 = jnp.zeros((LANES,), jnp.float32)
    for i in range(ZERO_PER_TILE // LANES):
        zeros_v[pl.ds(i * LANES, LANES)] = zeros16
    vals16 = jnp.full((LANES,), INV_TOTAL, jnp.float32)
    for i in range(IDX_PER_TILE // LANES):
        vals_v[pl.ds(i * LANES, LANES)] = vals16
    pltpu.sync_copy(zeros_v, hist_sh.at[pl.ds(sid * ZERO_PER_TILE, ZERO_PER_TILE)])
    idx_cp.wait()
    plsc.subcore_barrier()
    pltpu.sync_copy(vals_v, hist_sh.at[idx_row_v], add=True)
    plsc.subcore_barrier()
    goff = cid * BINS_PER_CORE + sid * BINS_PER_TILE
    out_v = zeros_v.at[pl.ds(0, BINS_PER_TILE)]
    pltpu.sync_copy(hist_sh.at[pl.ds(goff, BINS_PER_TILE)], out_v)
    pltpu.sync_copy(out_v, out_hbm.at[pl.ds(goff, BINS_PER_TILE)])


@functools.cache
def _hist():
    return functools.partial(
        pl.kernel,
        mesh=plsc.VectorSubcoreMesh(core_axis_name="c", subcore_axis_name="s"),
        out_type=jax.ShapeDtypeStruct((N_BINS_PAD,), jnp.float32),
        scratch_types=[
            pltpu.VMEM((IDX_PER_TILE,), jnp.int32),
            pltpu.VMEM((IDX_PER_TILE,), jnp.float32),
            pltpu.VMEM((ZERO_PER_TILE,), jnp.float32),
            pltpu.VMEM_SHARED((N_BINS_PAD,), jnp.float32),
            pltpu.SemaphoreType.DMA,
        ],
    )(_hist_body)


@jax.jit
def kernel(features):
    pooled, indices = _maxpool(features)
    hist = _hist()(indices.reshape(N_SUBCORES, IDX_PER_TILE))
    attention_weights = hist[:N_ROWS].reshape(1, N_ROWS)
    return (attention_weights, pooled)


# speedup vs baseline: 2.6743x; 1.0069x over previous
import functools

import jax
import jax.numpy as jnp
from jax import lax
from jax.experimental import pallas as pl
from jax.experimental.pallas import tpu as pltpu
from jax.experimental.pallas import tpu_sc as plsc

N_ROWS = 100000
N_COLS = 512
BLOCK_ROWS = 10000
NUM_BLOCKS = N_ROWS // BLOCK_ROWS

BINS_PER_TILE = 3136
N_BINS_PAD = 32 * BINS_PER_TILE
LANES = 16
INV_TOTAL = 1.0 / N_COLS

N_SUBCORES = 16
IDX_PER_TILE = N_COLS // N_SUBCORES
ZERO_PER_TILE = N_BINS_PAD // N_SUBCORES
BINS_PER_CORE = N_BINS_PAD // 2


def _maxpool_body(xl_ref, xr_ref, max_out, idx_out):
    i = pl.program_id(0)
    x = jnp.concatenate([xl_ref[...], xr_ref[...]], axis=1)
    bmax = jnp.max(x, axis=0, keepdims=True)
    rows = lax.broadcasted_iota(jnp.int32, x.shape, 0) + i * BLOCK_ROWS
    bidx = jnp.min(
        jnp.where(x == bmax, rows, jnp.int32(N_ROWS)), axis=0, keepdims=True
    )

    @pl.when(i == 0)
    def _():
        max_out[...] = bmax
        idx_out[...] = bidx

    @pl.when(i > 0)
    def _():
        better = bmax > max_out[...]
        idx_out[...] = jnp.where(better, bidx, idx_out[...])
        max_out[...] = jnp.where(better, bmax, max_out[...])


_maxpool = pl.pallas_call(
    _maxpool_body,
    grid=(NUM_BLOCKS,),
    in_specs=[
        pl.BlockSpec((BLOCK_ROWS, N_COLS // 2), lambda i: (i, 0)),
        pl.BlockSpec((BLOCK_ROWS, N_COLS // 2), lambda i: (i, 1)),
    ],
    out_specs=[
        pl.BlockSpec((1, N_COLS), lambda i: (0, 0)),
        pl.BlockSpec((1, N_COLS), lambda i: (0, 0)),
    ],
    out_shape=[
        jax.ShapeDtypeStruct((1, N_COLS), jnp.float32),
        jax.ShapeDtypeStruct((1, N_COLS), jnp.int32),
    ],
)


def _hist_body(idx_hbm, out_hbm, idx_row_v, vals_v, zeros_v, hist_sh, sem):
    cid = lax.axis_index("c")
    sid = lax.axis_index("s")
    idx_cp = pltpu.async_copy(idx_hbm.at[sid], idx_row_v, sem)
    zeros16 = jnp.zeros((LANES,), jnp.float32)
    for i in range(ZERO_PER_TILE // LANES):
        zeros_v[pl.ds(i * LANES, LANES)] = zeros16
    vals16 = jnp.full((LANES,), INV_TOTAL, jnp.float32)
    for i in range(IDX_PER_TILE // LANES):
        vals_v[pl.ds(i * LANES, LANES)] = vals16
    pltpu.sync_copy(zeros_v, hist_sh.at[pl.ds(sid * ZERO_PER_TILE, ZERO_PER_TILE)])
    idx_cp.wait()
    plsc.subcore_barrier()
    pltpu.sync_copy(vals_v, hist_sh.at[idx_row_v], add=True)
    plsc.subcore_barrier()
    goff = cid * BINS_PER_CORE + sid * BINS_PER_TILE
    out_v = zeros_v.at[pl.ds(0, BINS_PER_TILE)]
    pltpu.sync_copy(hist_sh.at[pl.ds(goff, BINS_PER_TILE)], out_v)
    pltpu.sync_copy(out_v, out_hbm.at[pl.ds(goff, BINS_PER_TILE)])


@functools.cache
def _hist():
    return functools.partial(
        pl.kernel,
        mesh=plsc.VectorSubcoreMesh(core_axis_name="c", subcore_axis_name="s"),
        out_type=jax.ShapeDtypeStruct((N_BINS_PAD,), jnp.float32),
        scratch_types=[
            pltpu.VMEM((IDX_PER_TILE,), jnp.int32),
            pltpu.VMEM((IDX_PER_TILE,), jnp.float32),
            pltpu.VMEM((ZERO_PER_TILE,), jnp.float32),
            pltpu.VMEM_SHARED((N_BINS_PAD,), jnp.float32),
            pltpu.SemaphoreType.DMA,
        ],
    )(_hist_body)


@jax.jit
def kernel(features):
    pooled, indices = _maxpool(features, features)
    hist = _hist()(indices.reshape(N_SUBCORES, IDX_PER_TILE))
    attention_weights = hist[:N_ROWS].reshape(1, N_ROWS)
    return (attention_weights, pooled)


# speedup vs baseline: 2.6932x; 1.0070x over previous
import functools

import jax
import jax.numpy as jnp
from jax import lax
from jax.experimental import pallas as pl
from jax.experimental.pallas import tpu as pltpu
from jax.experimental.pallas import tpu_sc as plsc

N_ROWS = 100000
N_COLS = 512
BLOCK_ROWS = 10000
NUM_BLOCKS = N_ROWS // BLOCK_ROWS

BINS_PER_TILE = 3136
N_BINS_PAD = 32 * BINS_PER_TILE
LANES = 16
INV_TOTAL = 1.0 / N_COLS

N_SUBCORES = 16
IDX_PER_TILE = N_COLS // N_SUBCORES
ZERO_PER_TILE = N_BINS_PAD // N_SUBCORES
BINS_PER_CORE = N_BINS_PAD // 2


def _maxpool_body(x0_ref, x1_ref, x2_ref, x3_ref, max_out, idx_out):
    i = pl.program_id(0)
    x = jnp.concatenate(
        [x0_ref[...], x1_ref[...], x2_ref[...], x3_ref[...]], axis=1
    )
    bmax = jnp.max(x, axis=0, keepdims=True)
    rows = lax.broadcasted_iota(jnp.int32, x.shape, 0) + i * BLOCK_ROWS
    bidx = jnp.min(
        jnp.where(x == bmax, rows, jnp.int32(N_ROWS)), axis=0, keepdims=True
    )

    @pl.when(i == 0)
    def _():
        max_out[...] = bmax
        idx_out[...] = bidx

    @pl.when(i > 0)
    def _():
        better = bmax > max_out[...]
        idx_out[...] = jnp.where(better, bidx, idx_out[...])
        max_out[...] = jnp.where(better, bmax, max_out[...])


_maxpool = pl.pallas_call(
    _maxpool_body,
    grid=(NUM_BLOCKS,),
    in_specs=[
        pl.BlockSpec((BLOCK_ROWS, N_COLS // 4), lambda i: (i, 0)),
        pl.BlockSpec((BLOCK_ROWS, N_COLS // 4), lambda i: (i, 1)),
        pl.BlockSpec((BLOCK_ROWS, N_COLS // 4), lambda i: (i, 2)),
        pl.BlockSpec((BLOCK_ROWS, N_COLS // 4), lambda i: (i, 3)),
    ],
    out_specs=[
        pl.BlockSpec((1, N_COLS), lambda i: (0, 0)),
        pl.BlockSpec((1, N_COLS), lambda i: (0, 0)),
    ],
    out_shape=[
        jax.ShapeDtypeStruct((1, N_COLS), jnp.float32),
        jax.ShapeDtypeStruct((1, N_COLS), jnp.int32),
    ],
)


def _hist_body(idx_hbm, out_hbm, idx_row_v, vals_v, zeros_v, hist_sh, sem):
    cid = lax.axis_index("c")
    sid = lax.axis_index("s")
    idx_cp = pltpu.async_copy(idx_hbm.at[sid], idx_row_v, sem)
    zeros16 = jnp.zeros((LANES,), jnp.float32)
    for i in range(ZERO_PER_TILE // LANES):
        zeros_v[pl.ds(i * LANES, LANES)] = zeros16
    vals16 = jnp.full((LANES,), INV_TOTAL, jnp.float32)
    for i in range(IDX_PER_TILE // LANES):
        vals_v[pl.ds(i * LANES, LANES)] = vals16
    pltpu.sync_copy(zeros_v, hist_sh.at[pl.ds(sid * ZERO_PER_TILE, ZERO_PER_TILE)])
    idx_cp.wait()
    plsc.subcore_barrier()
    pltpu.sync_copy(vals_v, hist_sh.at[idx_row_v], add=True)
    plsc.subcore_barrier()
    goff = cid * BINS_PER_CORE + sid * BINS_PER_TILE
    out_v = zeros_v.at[pl.ds(0, BINS_PER_TILE)]
    pltpu.sync_copy(hist_sh.at[pl.ds(goff, BINS_PER_TILE)], out_v)
    pltpu.sync_copy(out_v, out_hbm.at[pl.ds(goff, BINS_PER_TILE)])


@functools.cache
def _hist():
    return functools.partial(
        pl.kernel,
        mesh=plsc.VectorSubcoreMesh(core_axis_name="c", subcore_axis_name="s"),
        out_type=jax.ShapeDtypeStruct((N_BINS_PAD,), jnp.float32),
        scratch_types=[
            pltpu.VMEM((IDX_PER_TILE,), jnp.int32),
            pltpu.VMEM((IDX_PER_TILE,), jnp.float32),
            pltpu.VMEM((ZERO_PER_TILE,), jnp.float32),
            pltpu.VMEM_SHARED((N_BINS_PAD,), jnp.float32),
            pltpu.SemaphoreType.DMA,
        ],
    )(_hist_body)


@jax.jit
def kernel(features):
    pooled, indices = _maxpool(features, features, features, features)
    hist = _hist()(indices.reshape(N_SUBCORES, IDX_PER_TILE))
    attention_weights = hist[:N_ROWS].reshape(1, N_ROWS)
    return (attention_weights, pooled)


# speedup vs baseline: 2.7323x; 1.0145x over previous
import functools

import jax
import jax.numpy as jnp
from jax import lax
from jax.experimental import pallas as pl
from jax.experimental.pallas import tpu as pltpu
from jax.experimental.pallas import tpu_sc as plsc

N_ROWS = 100000
N_COLS = 512
BLOCK_ROWS = 10000
NUM_BLOCKS = N_ROWS // BLOCK_ROWS

BINS_PER_TILE = 3136
N_BINS_PAD = 32 * BINS_PER_TILE
LANES = 16
INV_TOTAL = 1.0 / N_COLS

N_SUBCORES = 16
IDX_PER_TILE = N_COLS // N_SUBCORES
ZERO_PER_TILE = N_BINS_PAD // N_SUBCORES
BINS_PER_CORE = N_BINS_PAD // 2


def _maxpool_body(x0_ref, x1_ref, x2_ref, x3_ref, max_out, idx_out):
    i = pl.program_id(0)
    x = jnp.concatenate(
        [x0_ref[...], x1_ref[...], x2_ref[...], x3_ref[...]], axis=1
    )
    bmax = jnp.max(x, axis=0, keepdims=True)
    rows = lax.broadcasted_iota(jnp.int32, x.shape, 0) + i * BLOCK_ROWS
    bidx = jnp.min(
        jnp.where(x == bmax, rows, jnp.int32(N_ROWS)), axis=0, keepdims=True
    )

    @pl.when(i == 0)
    def _():
        max_out[...] = bmax
        idx_out[...] = bidx

    @pl.when(i > 0)
    def _():
        better = bmax > max_out[...]
        idx_out[...] = jnp.where(better, bidx, idx_out[...])
        max_out[...] = jnp.where(better, bmax, max_out[...])


_maxpool = pl.pallas_call(
    _maxpool_body,
    grid=(NUM_BLOCKS,),
    in_specs=[
        pl.BlockSpec((BLOCK_ROWS, N_COLS // 4), lambda i: (i, 0)),
        pl.BlockSpec((BLOCK_ROWS, N_COLS // 4), lambda i: (i, 1)),
        pl.BlockSpec((BLOCK_ROWS, N_COLS // 4), lambda i: (i, 2)),
        pl.BlockSpec((BLOCK_ROWS, N_COLS // 4), lambda i: (i, 3)),
    ],
    out_specs=[
        pl.BlockSpec((1, N_COLS), lambda i: (0, 0)),
        pl.BlockSpec((1, N_COLS), lambda i: (0, 0)),
    ],
    out_shape=[
        jax.ShapeDtypeStruct((1, N_COLS), jnp.float32),
        jax.ShapeDtypeStruct((1, N_COLS), jnp.int32),
    ],
)


def _hist_body(idx_hbm, out_hbm, idx_v, hist_v, sem):
    cid = lax.axis_index("c")
    sid = lax.axis_index("s")
    wid = sid * 2 + cid
    base = wid * BINS_PER_TILE
    idx_cp = pltpu.async_copy(idx_hbm, idx_v, sem)
    zeros16 = jnp.zeros((LANES,), jnp.float32)
    for i in range(BINS_PER_TILE // LANES):
        hist_v[pl.ds(i * LANES, LANES)] = zeros16
    idx_cp.wait()
    vals16 = jnp.full((LANES,), INV_TOTAL, jnp.float32)
    for j in range(N_COLS // LANES):
        rel = idx_v[pl.ds(j * LANES, LANES)] - base
        mask = (rel >= 0) & (rel < BINS_PER_TILE)
        rel_c = jnp.clip(rel, 0, BINS_PER_TILE - 1)
        plsc.addupdate_scatter(hist_v, [rel_c], vals16, mask=mask)
    pltpu.sync_copy(hist_v, out_hbm.at[pl.ds(base, BINS_PER_TILE)])


@functools.cache
def _hist():
    return functools.partial(
        pl.kernel,
        mesh=plsc.VectorSubcoreMesh(core_axis_name="c", subcore_axis_name="s"),
        out_type=jax.ShapeDtypeStruct((N_BINS_PAD,), jnp.float32),
        scratch_types=[
            pltpu.VMEM((N_COLS,), jnp.int32),
            pltpu.VMEM((BINS_PER_TILE,), jnp.float32),
            pltpu.SemaphoreType.DMA,
        ],
        compiler_params=pltpu.CompilerParams(needs_layout_passes=False),
    )(_hist_body)


@jax.jit
def kernel(features):
    pooled, indices = _maxpool(features, features, features, features)
    hist = _hist()(indices.reshape(N_COLS))
    attention_weights = hist[:N_ROWS].reshape(1, N_ROWS)
    return (attention_weights, pooled)
